# Initial kernel scaffold; baseline (speedup 1.0000x reference)
import jax
import jax.numpy as jnp
from jax.experimental import pallas as pl


def kernel(input_ids, position_ids, W_tok, W_pd, b_pd, gamma, beta):
    raise NotImplementedError("write your pallas kernel here")


# speedup vs baseline: 1.5228x; 1.5228x over previous
import functools

import jax
import jax.numpy as jnp
from jax import lax
from jax.experimental import pallas as pl
from jax.experimental.pallas import tpu as pltpu
from jax.experimental.pallas import tpu_sc as plsc

SRC = 2048
BATCH = 4
HIDDEN = 768
ROWS = SRC * BATCH
NC, NS = 2, 16
NW = NC * NS
R_PER_W = ROWS // NW
CHUNK = 128


def _gather_sc(table, ids_flat):
    mesh = plsc.VectorSubcoreMesh(core_axis_name="c", subcore_axis_name="s")

    @functools.partial(
        pl.kernel,
        mesh=mesh,
        out_type=jax.ShapeDtypeStruct((ROWS, HIDDEN), jnp.float32),
        scratch_types=[
            pltpu.VMEM((CHUNK,), jnp.int32),
            pltpu.VMEM((CHUNK, HIDDEN), jnp.float32),
            pltpu.SemaphoreType.DMA,
        ],
    )
    def gather_kernel(table_hbm, idx_hbm, out_hbm, idx_v, rows_v, sem):
        wid = lax.axis_index("s") * NC + lax.axis_index("c")
        base = wid * R_PER_W
        for c in range(R_PER_W // CHUNK):
            off = base + c * CHUNK
            pltpu.sync_copy(idx_hbm.at[pl.ds(off, CHUNK)], idx_v)
            pltpu.async_copy(table_hbm.at[idx_v], rows_v, sem).wait()
            pltpu.sync_copy(rows_v, out_hbm.at[pl.ds(off, CHUNK)])

    return gather_kernel(table, ids_flat)


BLK = 512


def _tc_fused(tok_flat, pos_flat, w_t, b2, g2, bt2):
    def body(tok_ref, pos_ref, w_ref, b_ref, g_ref, bt_ref, out_ref):
        acc = jnp.dot(pos_ref[...], w_ref[...],
                      preferred_element_type=jnp.float32)
        p = 1.0 / (1.0 + jnp.exp(-(acc + b_ref[...])))
        e = tok_ref[...] + p
        mean = jnp.mean(e, axis=1, keepdims=True)
        cen = e - mean
        var = jnp.mean(cen * cen, axis=1, keepdims=True)
        out_ref[...] = cen * lax.rsqrt(var + 1e-5) * g_ref[...] + bt_ref[...]

    return pl.pallas_call(
        body,
        grid=(ROWS // BLK,),
        in_specs=[
            pl.BlockSpec((BLK, HIDDEN), lambda i: (i, 0)),
            pl.BlockSpec((BLK, HIDDEN), lambda i: (i, 0)),
            pl.BlockSpec((HIDDEN, HIDDEN), lambda i: (0, 0)),
            pl.BlockSpec((1, HIDDEN), lambda i: (0, 0)),
            pl.BlockSpec((1, HIDDEN), lambda i: (0, 0)),
            pl.BlockSpec((1, HIDDEN), lambda i: (0, 0)),
        ],
        out_specs=pl.BlockSpec((BLK, HIDDEN), lambda i: (i, 0)),
        out_shape=jax.ShapeDtypeStruct((ROWS, HIDDEN), jnp.float32),
    )(tok_flat, pos_flat, w_t, b2, g2, bt2)


def kernel(input_ids, position_ids, W_tok, W_pd, b_pd, gamma, beta):
    ids_flat = input_ids.reshape(ROWS).astype(jnp.int32)
    tok_flat = _gather_sc(W_tok, ids_flat)
    pos_flat = position_ids.reshape(ROWS, HIDDEN)
    out_flat = _tc_fused(
        tok_flat, pos_flat, W_pd.T,
        b_pd.reshape(1, HIDDEN), gamma.reshape(1, HIDDEN),
        beta.reshape(1, HIDDEN),
    )
    return out_flat.reshape(SRC, BATCH, HIDDEN)


# speedup vs baseline: 2.4802x; 1.6288x over previous
import functools

import jax
import jax.numpy as jnp
from jax import lax
from jax.experimental import pallas as pl
from jax.experimental.pallas import tpu as pltpu
from jax.experimental.pallas import tpu_sc as plsc

SRC = 2048
BATCH = 4
HIDDEN = 768
ROWS = SRC * BATCH
NC, NS = 2, 16
NW = NC * NS
R_PER_W = ROWS // NW
CHUNK = 128


def _gather_sc(table, ids_flat):
    mesh = plsc.VectorSubcoreMesh(core_axis_name="c", subcore_axis_name="s")

    @functools.partial(
        pl.kernel,
        mesh=mesh,
        out_type=jax.ShapeDtypeStruct((ROWS, HIDDEN), jnp.float32),
        scratch_types=[
            pltpu.VMEM((CHUNK,), jnp.int32),
            pltpu.VMEM((CHUNK, HIDDEN), jnp.float32),
            pltpu.SemaphoreType.DMA,
        ],
    )
    def gather_kernel(table_hbm, idx_hbm, out_hbm, idx_v, rows_v, sem):
        wid = lax.axis_index("s") * NC + lax.axis_index("c")
        base = wid * R_PER_W
        for c in range(R_PER_W // CHUNK):
            off = base + c * CHUNK
            pltpu.sync_copy(idx_hbm.at[pl.ds(off, CHUNK)], idx_v)
            pltpu.async_copy(table_hbm.at[idx_v], rows_v, sem).wait()
            pltpu.sync_copy(rows_v, out_hbm.at[pl.ds(off, CHUNK)])

    return gather_kernel(table, ids_flat)


BS_S = 128
BLK = BS_S * BATCH


def _tc_fused(tok_flat, pos3, w_t, b2, g2, bt2):
    def body(tok_ref, pos_ref, w_ref, b_ref, g_ref, bt_ref, out_ref):
        pos = pos_ref[...].reshape(BLK, HIDDEN)
        acc = jnp.dot(pos, w_ref[...], preferred_element_type=jnp.float32)
        p = 1.0 / (1.0 + jnp.exp(-(acc + b_ref[...])))
        e = tok_ref[...] + p
        mean = jnp.mean(e, axis=1, keepdims=True)
        cen = e - mean
        var = jnp.mean(cen * cen, axis=1, keepdims=True)
        res = cen * lax.rsqrt(var + 1e-5) * g_ref[...] + bt_ref[...]
        out_ref[...] = res.reshape(BS_S, BATCH, HIDDEN)

    return pl.pallas_call(
        body,
        grid=(SRC // BS_S,),
        in_specs=[
            pl.BlockSpec((BLK, HIDDEN), lambda i: (i, 0)),
            pl.BlockSpec((BS_S, BATCH, HIDDEN), lambda i: (i, 0, 0)),
            pl.BlockSpec((HIDDEN, HIDDEN), lambda i: (0, 0)),
            pl.BlockSpec((1, HIDDEN), lambda i: (0, 0)),
            pl.BlockSpec((1, HIDDEN), lambda i: (0, 0)),
            pl.BlockSpec((1, HIDDEN), lambda i: (0, 0)),
        ],
        out_specs=pl.BlockSpec((BS_S, BATCH, HIDDEN), lambda i: (i, 0, 0)),
        out_shape=jax.ShapeDtypeStruct((SRC, BATCH, HIDDEN), jnp.float32),
    )(tok_flat, pos3, w_t, b2, g2, bt2)


def kernel(input_ids, position_ids, W_tok, W_pd, b_pd, gamma, beta):
    ids_flat = input_ids.reshape(ROWS).astype(jnp.int32)
    tok_flat = _gather_sc(W_tok, ids_flat)
    return _tc_fused(
        tok_flat, position_ids, W_pd.T,
        b_pd.reshape(1, HIDDEN), gamma.reshape(1, HIDDEN),
        beta.reshape(1, HIDDEN),
    )
